# baseline (speedup 1.0000x reference)
import functools

import jax
import jax.numpy as jnp
from jax import lax
from jax.experimental import pallas as pl
from jax.experimental.pallas import tpu as pltpu
from jax.experimental.pallas import tpu_sc as plsc

_NC = 2
_NS = 16
_NW = _NC * _NS
_LANES = 16
_CHUNK_R = 8
_NBUF = 6
_UNROLL = 4
_SC_ROWS = 2048
_TC_BLOCK_R = 512


def _make_sc_add(sc_rows, d):
    rows_per_w = sc_rows // _NW
    n_chunks = rows_per_w // _CHUNK_R
    mesh = plsc.VectorSubcoreMesh(core_axis_name="c", subcore_axis_name="s")

    @functools.partial(
        pl.kernel,
        mesh=mesh,
        out_type=jax.ShapeDtypeStruct((sc_rows, d), jnp.float32),
        scratch_types=[
            pltpu.VMEM((_NBUF, _CHUNK_R, d), jnp.float32),
            pltpu.VMEM((_NBUF, _CHUNK_R, d), jnp.float32),
        ]
        + [pltpu.SemaphoreType.DMA] * _NBUF
        + [pltpu.SemaphoreType.DMA] * _NBUF
        + [pltpu.SemaphoreType.DMA] * _NBUF,
    )
    def sc_add(x_hbm, p_hbm, o_hbm, xbuf, pbuf, *sems):
        xl_sem = sems[0:_NBUF]
        pl_sem = sems[_NBUF:2 * _NBUF]
        st_sem = sems[2 * _NBUF:3 * _NBUF]
        wid = lax.axis_index("s") * _NC + lax.axis_index("c")
        base_row = wid * rows_per_w

        def start_loads(c):
            b = c % _NBUF
            row = base_row + c * _CHUNK_R
            pltpu.async_copy(x_hbm.at[pl.ds(row, _CHUNK_R)], xbuf.at[b],
                             xl_sem[b])
            pltpu.async_copy(p_hbm.at[pl.ds(row, _CHUNK_R)], pbuf.at[b],
                             pl_sem[b])

        for c in range(min(_NBUF - 1, n_chunks)):
            start_loads(c)

        for c in range(n_chunks):
            b = c % _NBUF
            row = base_row + c * _CHUNK_R
            pltpu.make_async_copy(x_hbm.at[pl.ds(row, _CHUNK_R)], xbuf.at[b],
                                  xl_sem[b]).wait()
            pltpu.make_async_copy(p_hbm.at[pl.ds(row, _CHUNK_R)], pbuf.at[b],
                                  pl_sem[b]).wait()

            def vbody(j, b=b):
                s = pl.ds(j, _LANES)
                for r in range(_CHUNK_R):
                    plsc.addupdate(xbuf.at[b, r, s], pbuf[b, r, s])

            plsc.parallel_loop(0, d, step=_LANES, unroll=_UNROLL)(vbody)

            pltpu.async_copy(xbuf.at[b], o_hbm.at[pl.ds(row, _CHUNK_R)],
                             st_sem[b])
            f = c + _NBUF - 1
            if f < n_chunks:
                fb = f % _NBUF
                if f >= _NBUF:
                    frow = base_row + (f - _NBUF) * _CHUNK_R
                    pltpu.make_async_copy(
                        xbuf.at[fb], o_hbm.at[pl.ds(frow, _CHUNK_R)],
                        st_sem[fb]).wait()
                start_loads(f)

        for c in range(max(0, n_chunks - _NBUF), n_chunks):
            b = c % _NBUF
            row = base_row + c * _CHUNK_R
            pltpu.make_async_copy(xbuf.at[b], o_hbm.at[pl.ds(row, _CHUNK_R)],
                                  st_sem[b]).wait()

    return sc_add


def _tc_add_body(x_ref, p_ref, o_ref):
    o_ref[...] = x_ref[...] + p_ref[...]


def _tc_add(x, position_matrix, row0, rows):
    n, d = x.shape
    blk0 = row0 // _TC_BLOCK_R
    in_spec = pl.BlockSpec((_TC_BLOCK_R, d), lambda i: (i + blk0, 0))
    out_spec = pl.BlockSpec((_TC_BLOCK_R, d), lambda i: (i, 0))
    return pl.pallas_call(
        _tc_add_body,
        grid=(rows // _TC_BLOCK_R,),
        in_specs=[in_spec, in_spec],
        out_specs=out_spec,
        out_shape=jax.ShapeDtypeStruct((rows, d), x.dtype),
    )(x, position_matrix)


def kernel(x, position_matrix):
    n, d = x.shape
    sc_part = _make_sc_add(_SC_ROWS, d)(x, position_matrix)
    tc_part = _tc_add(x, position_matrix, _SC_ROWS, n - _SC_ROWS)
    return jnp.concatenate([sc_part, tc_part], axis=0)


# speedup vs baseline: 1.1264x; 1.1264x over previous
import functools

import jax
import jax.numpy as jnp
from jax import lax
from jax.experimental import pallas as pl
from jax.experimental.pallas import tpu as pltpu
from jax.experimental.pallas import tpu_sc as plsc

_NC = 2
_NS = 16
_NW = _NC * _NS
_LANES = 16
_CHUNK_R = 8
_NBUF = 6
_UNROLL = 4


def _make_sc_add(n, d):
    rows_per_w = n // _NW
    n_chunks = rows_per_w // _CHUNK_R
    mesh = plsc.VectorSubcoreMesh(core_axis_name="c", subcore_axis_name="s")

    @functools.partial(
        pl.kernel,
        mesh=mesh,
        out_type=jax.ShapeDtypeStruct((n, d), jnp.float32),
        scratch_types=[
            pltpu.VMEM((_NBUF, _CHUNK_R, d), jnp.float32),
            pltpu.VMEM((_NBUF, _CHUNK_R, d), jnp.float32),
        ]
        + [pltpu.SemaphoreType.DMA] * _NBUF
        + [pltpu.SemaphoreType.DMA] * _NBUF
        + [pltpu.SemaphoreType.DMA] * _NBUF,
    )
    def sc_add(x_hbm, p_hbm, o_hbm, xbuf, pbuf, *sems):
        xl_sem = sems[0:_NBUF]
        pl_sem = sems[_NBUF:2 * _NBUF]
        st_sem = sems[2 * _NBUF:3 * _NBUF]
        wid = lax.axis_index("s") * _NC + lax.axis_index("c")
        base_row = wid * rows_per_w

        def start_loads(c):
            b = c % _NBUF
            row = base_row + c * _CHUNK_R
            pltpu.async_copy(x_hbm.at[pl.ds(row, _CHUNK_R)], xbuf.at[b],
                             xl_sem[b])
            pltpu.async_copy(p_hbm.at[pl.ds(row, _CHUNK_R)], pbuf.at[b],
                             pl_sem[b])

        for c in range(min(_NBUF - 1, n_chunks)):
            start_loads(c)

        for c in range(n_chunks):
            b = c % _NBUF
            row = base_row + c * _CHUNK_R
            pltpu.make_async_copy(x_hbm.at[pl.ds(row, _CHUNK_R)], xbuf.at[b],
                                  xl_sem[b]).wait()
            pltpu.make_async_copy(p_hbm.at[pl.ds(row, _CHUNK_R)], pbuf.at[b],
                                  pl_sem[b]).wait()

            def vbody(j, b=b):
                s = pl.ds(j, _LANES)
                for r in range(_CHUNK_R):
                    plsc.addupdate(xbuf.at[b, r, s], pbuf[b, r, s])

            plsc.parallel_loop(0, d, step=_LANES, unroll=_UNROLL)(vbody)

            pltpu.async_copy(xbuf.at[b], o_hbm.at[pl.ds(row, _CHUNK_R)],
                             st_sem[b])
            f = c + _NBUF - 1
            if f < n_chunks:
                fb = f % _NBUF
                if f >= _NBUF:
                    frow = base_row + (f - _NBUF) * _CHUNK_R
                    pltpu.make_async_copy(
                        xbuf.at[fb], o_hbm.at[pl.ds(frow, _CHUNK_R)],
                        st_sem[fb]).wait()
                start_loads(f)

        for c in range(max(0, n_chunks - _NBUF), n_chunks):
            b = c % _NBUF
            row = base_row + c * _CHUNK_R
            pltpu.make_async_copy(xbuf.at[b], o_hbm.at[pl.ds(row, _CHUNK_R)],
                                  st_sem[b]).wait()

    return sc_add


def kernel(x, position_matrix):
    n, d = x.shape
    return _make_sc_add(n, d)(x, position_matrix)


# speedup vs baseline: 1.2687x; 1.1264x over previous
import functools

import jax
import jax.numpy as jnp
from jax import lax
from jax.experimental import pallas as pl
from jax.experimental.pallas import tpu as pltpu
from jax.experimental.pallas import tpu_sc as plsc

_NC = 2
_NS = 16
_NW = _NC * _NS
_LANES = 16
_CHUNK_R = 8
_NBUF = 4
_UNROLL = 4


def _make_sc_add(n, d):
    rows_per_w = n // _NW
    n_chunks = rows_per_w // _CHUNK_R
    n_groups = (n_chunks - _NBUF) // _NBUF
    assert n_chunks == (n_groups + 1) * _NBUF, (n_chunks, n_groups)
    mesh = plsc.VectorSubcoreMesh(core_axis_name="c", subcore_axis_name="s")

    @functools.partial(
        pl.kernel,
        mesh=mesh,
        out_type=jax.ShapeDtypeStruct((n, d), jnp.float32),
        scratch_types=[
            pltpu.VMEM((_NBUF, _CHUNK_R, d), jnp.float32),
            pltpu.VMEM((_NBUF, _CHUNK_R, d), jnp.float32),
        ]
        + [pltpu.SemaphoreType.DMA] * _NBUF
        + [pltpu.SemaphoreType.DMA] * _NBUF
        + [pltpu.SemaphoreType.DMA] * _NBUF,
    )
    def sc_add(x_hbm, p_hbm, o_hbm, xbuf, pbuf, *sems):
        xl_sem = sems[0:_NBUF]
        pl_sem = sems[_NBUF:2 * _NBUF]
        st_sem = sems[2 * _NBUF:3 * _NBUF]
        wid = lax.axis_index("s") * _NC + lax.axis_index("c")
        base_row = wid * rows_per_w

        def start_loads(c, b):
            row = base_row + c * _CHUNK_R
            pltpu.async_copy(x_hbm.at[pl.ds(row, _CHUNK_R)], xbuf.at[b],
                             xl_sem[b])
            pltpu.async_copy(p_hbm.at[pl.ds(row, _CHUNK_R)], pbuf.at[b],
                             pl_sem[b])

        def wait_loads(c, b):
            row = base_row + c * _CHUNK_R
            pltpu.make_async_copy(x_hbm.at[pl.ds(row, _CHUNK_R)], xbuf.at[b],
                                  xl_sem[b]).wait()
            pltpu.make_async_copy(p_hbm.at[pl.ds(row, _CHUNK_R)], pbuf.at[b],
                                  pl_sem[b]).wait()

        def compute(b):
            def vbody(j, b=b):
                s = pl.ds(j, _LANES)
                for r in range(_CHUNK_R):
                    plsc.addupdate(xbuf.at[b, r, s], pbuf[b, r, s])

            plsc.parallel_loop(0, d, step=_LANES, unroll=_UNROLL)(vbody)

        def start_store(c, b):
            row = base_row + c * _CHUNK_R
            pltpu.async_copy(xbuf.at[b], o_hbm.at[pl.ds(row, _CHUNK_R)],
                             st_sem[b])

        def wait_store(c, b):
            row = base_row + c * _CHUNK_R
            pltpu.make_async_copy(xbuf.at[b], o_hbm.at[pl.ds(row, _CHUNK_R)],
                                  st_sem[b]).wait()

        for c in range(_NBUF - 1):
            start_loads(c, c)

        wait_loads(0, 0)
        compute(0)
        start_store(0, 0)
        start_loads(_NBUF - 1, _NBUF - 1)

        def group_body(g, carry):
            c0 = 1 + g * _NBUF
            for i in range(_NBUF):
                c = c0 + i
                b = (1 + i) % _NBUF
                wait_loads(c, b)
                compute(b)
                start_store(c, b)
                fb = i % _NBUF
                wait_store(c - 1, fb)
                start_loads(c + _NBUF - 1, fb)
            return carry

        lax.fori_loop(0, n_groups, group_body, 0)

        for c in range(n_chunks - _NBUF + 1, n_chunks):
            b = c % _NBUF
            wait_loads(c, b)
            compute(b)
            start_store(c, b)

        for c in range(n_chunks - _NBUF, n_chunks):
            wait_store(c, c % _NBUF)

    return sc_add


def kernel(x, position_matrix):
    n, d = x.shape
    return _make_sc_add(n, d)(x, position_matrix)


# speedup vs baseline: 1.2900x; 1.0168x over previous
import functools

import jax
import jax.numpy as jnp
from jax import lax
from jax.experimental import pallas as pl
from jax.experimental.pallas import tpu as pltpu
from jax.experimental.pallas import tpu_sc as plsc

_NC = 2
_NS = 16
_NW = _NC * _NS
_LANES = 16
_CHUNK_R = 8
_NBUF = 4
_UNROLL = 2


def _make_sc_add(n, d):
    rows_per_w = n // _NW
    n_chunks = rows_per_w // _CHUNK_R
    n_groups = (n_chunks - _NBUF) // _NBUF
    assert n_chunks == (n_groups + 1) * _NBUF, (n_chunks, n_groups)
    mesh = plsc.VectorSubcoreMesh(core_axis_name="c", subcore_axis_name="s")

    @functools.partial(
        pl.kernel,
        mesh=mesh,
        out_type=jax.ShapeDtypeStruct((n, d), jnp.float32),
        scratch_types=[
            pltpu.VMEM((_NBUF, _CHUNK_R, d), jnp.float32),
            pltpu.VMEM((_NBUF, _CHUNK_R, d), jnp.float32),
        ]
        + [pltpu.SemaphoreType.DMA] * _NBUF
        + [pltpu.SemaphoreType.DMA] * _NBUF
        + [pltpu.SemaphoreType.DMA] * _NBUF,
    )
    def sc_add(x_hbm, p_hbm, o_hbm, xbuf, pbuf, *sems):
        xl_sem = sems[0:_NBUF]
        pl_sem = sems[_NBUF:2 * _NBUF]
        st_sem = sems[2 * _NBUF:3 * _NBUF]
        wid = lax.axis_index("s") * _NC + lax.axis_index("c")
        base_row = wid * rows_per_w

        def start_loads(c, b):
            row = base_row + c * _CHUNK_R
            pltpu.async_copy(x_hbm.at[pl.ds(row, _CHUNK_R)], xbuf.at[b],
                             xl_sem[b])
            pltpu.async_copy(p_hbm.at[pl.ds(row, _CHUNK_R)], pbuf.at[b],
                             pl_sem[b])

        def wait_loads(c, b):
            row = base_row + c * _CHUNK_R
            pltpu.make_async_copy(x_hbm.at[pl.ds(row, _CHUNK_R)], xbuf.at[b],
                                  xl_sem[b]).wait()
            pltpu.make_async_copy(p_hbm.at[pl.ds(row, _CHUNK_R)], pbuf.at[b],
                                  pl_sem[b]).wait()

        def compute(b):
            def vbody(j, b=b):
                s = pl.ds(j, _LANES)
                for r in range(_CHUNK_R):
                    plsc.addupdate(xbuf.at[b, r, s], pbuf[b, r, s])

            plsc.parallel_loop(0, d, step=_LANES, unroll=_UNROLL)(vbody)

        def start_store(c, b):
            row = base_row + c * _CHUNK_R
            pltpu.async_copy(xbuf.at[b], o_hbm.at[pl.ds(row, _CHUNK_R)],
                             st_sem[b])

        def wait_store(c, b):
            row = base_row + c * _CHUNK_R
            pltpu.make_async_copy(xbuf.at[b], o_hbm.at[pl.ds(row, _CHUNK_R)],
                                  st_sem[b]).wait()

        for c in range(_NBUF - 1):
            start_loads(c, c)

        wait_loads(0, 0)
        compute(0)
        start_store(0, 0)
        start_loads(_NBUF - 1, _NBUF - 1)

        def group_body(g, carry):
            c0 = 1 + g * _NBUF
            for i in range(_NBUF):
                c = c0 + i
                b = (1 + i) % _NBUF
                wait_loads(c, b)
                compute(b)
                start_store(c, b)
                fb = i % _NBUF
                wait_store(c - 1, fb)
                start_loads(c + _NBUF - 1, fb)
            return carry

        lax.fori_loop(0, n_groups, group_body, 0)

        for c in range(n_chunks - _NBUF + 1, n_chunks):
            b = c % _NBUF
            wait_loads(c, b)
            compute(b)
            start_store(c, b)

        for c in range(n_chunks - _NBUF, n_chunks):
            wait_store(c, c % _NBUF)

    return sc_add


def kernel(x, position_matrix):
    n, d = x.shape
    return _make_sc_add(n, d)(x, position_matrix)


# speedup vs baseline: 1.3099x; 1.0154x over previous
import functools

import jax
import jax.numpy as jnp
from jax import lax
from jax.experimental import pallas as pl
from jax.experimental.pallas import tpu as pltpu
from jax.experimental.pallas import tpu_sc as plsc

_NC = 2
_NS = 16
_NW = _NC * _NS
_LANES = 16
_CHUNK_R = 8
_NBUF = 4
_UNROLL = 1


def _make_sc_add(n, d):
    rows_per_w = n // _NW
    n_chunks = rows_per_w // _CHUNK_R
    n_groups = (n_chunks - _NBUF) // _NBUF
    assert n_chunks == (n_groups + 1) * _NBUF, (n_chunks, n_groups)
    mesh = plsc.VectorSubcoreMesh(core_axis_name="c", subcore_axis_name="s")

    @functools.partial(
        pl.kernel,
        mesh=mesh,
        out_type=jax.ShapeDtypeStruct((n, d), jnp.float32),
        scratch_types=[
            pltpu.VMEM((_NBUF, _CHUNK_R, d), jnp.float32),
            pltpu.VMEM((_NBUF, _CHUNK_R, d), jnp.float32),
        ]
        + [pltpu.SemaphoreType.DMA] * _NBUF
        + [pltpu.SemaphoreType.DMA] * _NBUF
        + [pltpu.SemaphoreType.DMA] * _NBUF,
    )
    def sc_add(x_hbm, p_hbm, o_hbm, xbuf, pbuf, *sems):
        xl_sem = sems[0:_NBUF]
        pl_sem = sems[_NBUF:2 * _NBUF]
        st_sem = sems[2 * _NBUF:3 * _NBUF]
        wid = lax.axis_index("s") * _NC + lax.axis_index("c")
        base_row = wid * rows_per_w

        def start_loads(c, b):
            row = base_row + c * _CHUNK_R
            pltpu.async_copy(x_hbm.at[pl.ds(row, _CHUNK_R)], xbuf.at[b],
                             xl_sem[b])
            pltpu.async_copy(p_hbm.at[pl.ds(row, _CHUNK_R)], pbuf.at[b],
                             pl_sem[b])

        def wait_loads(c, b):
            row = base_row + c * _CHUNK_R
            pltpu.make_async_copy(x_hbm.at[pl.ds(row, _CHUNK_R)], xbuf.at[b],
                                  xl_sem[b]).wait()
            pltpu.make_async_copy(p_hbm.at[pl.ds(row, _CHUNK_R)], pbuf.at[b],
                                  pl_sem[b]).wait()

        def compute(b):
            def vbody(j, b=b):
                s = pl.ds(j, _LANES)
                for r in range(_CHUNK_R):
                    plsc.addupdate(xbuf.at[b, r, s], pbuf[b, r, s])

            plsc.parallel_loop(0, d, step=_LANES, unroll=_UNROLL)(vbody)

        def start_store(c, b):
            row = base_row + c * _CHUNK_R
            pltpu.async_copy(xbuf.at[b], o_hbm.at[pl.ds(row, _CHUNK_R)],
                             st_sem[b])

        def wait_store(c, b):
            row = base_row + c * _CHUNK_R
            pltpu.make_async_copy(xbuf.at[b], o_hbm.at[pl.ds(row, _CHUNK_R)],
                                  st_sem[b]).wait()

        for c in range(_NBUF):
            start_loads(c, c)

        wait_loads(0, 0)
        compute(0)
        start_store(0, 0)

        def group_body(g, carry):
            c0 = 1 + g * _NBUF
            for i in range(_NBUF):
                c = c0 + i
                b = (1 + i) % _NBUF
                wait_loads(c, b)
                compute(b)
                start_store(c, b)
                fb = i % _NBUF
                wait_store(c - 1, fb)
                start_loads(c + _NBUF - 1, fb)
            return carry

        lax.fori_loop(0, n_groups, group_body, 0)

        for c in range(n_chunks - _NBUF + 1, n_chunks):
            b = c % _NBUF
            wait_loads(c, b)
            compute(b)
            start_store(c, b)

        for c in range(n_chunks - _NBUF, n_chunks):
            wait_store(c, c % _NBUF)

    return sc_add


def kernel(x, position_matrix):
    n, d = x.shape
    return _make_sc_add(n, d)(x, position_matrix)
